# baseline (speedup 1.0000x reference)
import functools

import jax
import jax.numpy as jnp
from jax import lax
from jax.experimental import pallas as pl
from jax.experimental.pallas import tpu as pltpu
from jax.experimental.pallas import tpu_sc as plsc

_MAX_DEGREE = 512
_HIDDEN = 128
_N = 100000

_NC = 2
_NS = 16
_NW = _NC * _NS

_CHUNK = 448
_FULL = 7 * _CHUNK
_TAIL_BASE = 31 * _FULL + 6 * _CHUNK
_TAIL = _N - _TAIL_BASE


def _body(deg_hbm, table_hbm, out_hbm,
          idx0, idx1, rows0, rows1, idx_t, rows_t,
          gsem0, gsem1, wsem0, wsem1, tgsem, twsem):
    c = lax.axis_index("c")
    s = lax.axis_index("s")
    wid = s * _NC + c
    base = wid * _FULL

    idx = [idx0, idx1]
    rows = [rows0, rows1]
    gsem = [gsem0, gsem1]
    wsem = [wsem0, wsem1]

    def clamp(idxb, n):
        for i in range(n // 16):
            sl = pl.ds(i * 16, 16)
            v = idxb[sl]
            idxb[sl] = jnp.minimum(jnp.maximum(v, 0), _MAX_DEGREE - 1)

    def do_chunk(off, b):
        pltpu.sync_copy(deg_hbm.at[pl.ds(off, _CHUNK)], idx[b])
        clamp(idx[b], _CHUNK)
        pltpu.async_copy(table_hbm.at[idx[b]], rows[b], gsem[b]).wait()
        return pltpu.async_copy(rows[b], out_hbm.at[pl.ds(off, _CHUNK)], wsem[b])

    wh = [None, None]
    for j in range(6):
        b = j & 1
        if wh[b] is not None:
            wh[b].wait()
        wh[b] = do_chunk(base + j * _CHUNK, b)

    @pl.when(wid < _NW - 1)
    def _():
        wh[0].wait()
        do_chunk(base + 6 * _CHUNK, 0).wait()
        wh[1].wait()

    @pl.when(wid == _NW - 1)
    def _():
        pltpu.sync_copy(deg_hbm.at[pl.ds(_TAIL_BASE, _TAIL)], idx_t)
        clamp(idx_t, _TAIL)
        pltpu.async_copy(table_hbm.at[idx_t], rows_t, tgsem).wait()
        pltpu.async_copy(rows_t, out_hbm.at[pl.ds(_TAIL_BASE, _TAIL)], twsem).wait()
        wh[0].wait()
        wh[1].wait()


@jax.jit
def _run(degrees, table):
    mesh = plsc.VectorSubcoreMesh(core_axis_name="c", subcore_axis_name="s")
    k = pl.kernel(
        _body,
        mesh=mesh,
        out_type=jax.ShapeDtypeStruct((_N, _HIDDEN), jnp.float32),
        scratch_types=[
            pltpu.VMEM((_CHUNK,), jnp.int32),
            pltpu.VMEM((_CHUNK,), jnp.int32),
            pltpu.VMEM((_CHUNK, _HIDDEN), jnp.float32),
            pltpu.VMEM((_CHUNK, _HIDDEN), jnp.float32),
            pltpu.VMEM((_TAIL,), jnp.int32),
            pltpu.VMEM((_TAIL, _HIDDEN), jnp.float32),
            pltpu.SemaphoreType.DMA,
            pltpu.SemaphoreType.DMA,
            pltpu.SemaphoreType.DMA,
            pltpu.SemaphoreType.DMA,
            pltpu.SemaphoreType.DMA,
            pltpu.SemaphoreType.DMA,
        ],
    )
    return k(degrees, table)


def kernel(degrees, degree_embedding):
    return _run(degrees.astype(jnp.int32), degree_embedding)


# speedup vs baseline: 1.0121x; 1.0121x over previous
import jax
import jax.numpy as jnp
from jax import lax
from jax.experimental import pallas as pl
from jax.experimental.pallas import tpu as pltpu
from jax.experimental.pallas import tpu_sc as plsc

_MAX_DEGREE = 512
_HIDDEN = 128
_N = 100000

_NC = 2
_NS = 16
_NW = _NC * _NS

_CHUNK = 224
_NBUF = 4
_FULL = 3136
_NCH_FULL = _FULL // _CHUNK
_LAST = 2784
_NCH_LAST = 12
_TAIL_BASE = 31 * _FULL + _NCH_LAST * _CHUNK
_TAIL = _N - _TAIL_BASE


def _body(deg_hbm, table_hbm, out_hbm,
          idxa, b0, b1, b2, b3, idx_t, rows_t,
          g0, g1, g2, g3, w0, w1, w2, w3, tg, tw):
    c = lax.axis_index("c")
    s = lax.axis_index("s")
    wid = s * _NC + c
    base = wid * _FULL

    bufs = [b0, b1, b2, b3]
    gsem = [g0, g1, g2, g3]
    wsem = [w0, w1, w2, w3]

    def load_and_clamp(n):
        pltpu.sync_copy(deg_hbm.at[pl.ds(base, n)], idxa.at[pl.ds(0, n)])
        for i in range(n // 16):
            sl = pl.ds(i * 16, 16)
            idxa[sl] = jnp.minimum(jnp.maximum(idxa[sl], 0), _MAX_DEGREE - 1)

    def pipe(nch):
        def fire_gather(j, b):
            return pltpu.async_copy(
                table_hbm.at[idxa.at[pl.ds(j * _CHUNK, _CHUNK)]],
                bufs[b], gsem[b])

        def fire_write(j, b):
            return pltpu.async_copy(
                bufs[b], out_hbm.at[pl.ds(base + j * _CHUNK, _CHUNK)], wsem[b])

        gh = [None] * _NBUF
        wh = [None] * _NBUF
        for j in range(min(_NBUF, nch)):
            gh[j] = fire_gather(j, j)
        for j in range(nch):
            b = j % _NBUF
            gh[b].wait()
            wh[b] = fire_write(j, b)
            jn = j + _NBUF
            if jn < nch:
                wh[b].wait()
                gh[b] = fire_gather(jn, b)
        for j in range(max(nch - _NBUF, 0), nch):
            wh[j % _NBUF].wait()

    @pl.when(wid < _NW - 1)
    def _():
        load_and_clamp(_FULL)
        pipe(_NCH_FULL)

    @pl.when(wid == _NW - 1)
    def _():
        load_and_clamp(_LAST)
        pipe(_NCH_LAST)
        pltpu.sync_copy(deg_hbm.at[pl.ds(_TAIL_BASE, _TAIL)], idx_t)
        for i in range(_TAIL // 16):
            sl = pl.ds(i * 16, 16)
            idx_t[sl] = jnp.minimum(jnp.maximum(idx_t[sl], 0), _MAX_DEGREE - 1)
        pltpu.async_copy(table_hbm.at[idx_t], rows_t, tg).wait()
        pltpu.async_copy(rows_t, out_hbm.at[pl.ds(_TAIL_BASE, _TAIL)], tw).wait()


@jax.jit
def _run(degrees, table):
    mesh = plsc.VectorSubcoreMesh(core_axis_name="c", subcore_axis_name="s")
    k = pl.kernel(
        _body,
        mesh=mesh,
        out_type=jax.ShapeDtypeStruct((_N, _HIDDEN), jnp.float32),
        scratch_types=[
            pltpu.VMEM((_FULL,), jnp.int32),
            pltpu.VMEM((_CHUNK, _HIDDEN), jnp.float32),
            pltpu.VMEM((_CHUNK, _HIDDEN), jnp.float32),
            pltpu.VMEM((_CHUNK, _HIDDEN), jnp.float32),
            pltpu.VMEM((_CHUNK, _HIDDEN), jnp.float32),
            pltpu.VMEM((_TAIL,), jnp.int32),
            pltpu.VMEM((_TAIL, _HIDDEN), jnp.float32),
            pltpu.SemaphoreType.DMA,
            pltpu.SemaphoreType.DMA,
            pltpu.SemaphoreType.DMA,
            pltpu.SemaphoreType.DMA,
            pltpu.SemaphoreType.DMA,
            pltpu.SemaphoreType.DMA,
            pltpu.SemaphoreType.DMA,
            pltpu.SemaphoreType.DMA,
            pltpu.SemaphoreType.DMA,
            pltpu.SemaphoreType.DMA,
        ],
    )
    return k(degrees, table)


def kernel(degrees, degree_embedding):
    return _run(degrees.astype(jnp.int32), degree_embedding)
